# baseline (speedup 1.0000x reference)
import functools

import jax
import jax.numpy as jnp
from jax import lax
from jax.experimental import pallas as pl
from jax.experimental.pallas import tpu as pltpu
from jax.experimental.pallas import tpu_sc as plsc

_H = 384
_W = 384
_N = _H * _W
_K = int(_N * 0.15)
_START_WARM = 20000

_MESH = plsc.VectorSubcoreMesh(core_axis_name="c", subcore_axis_name="s")
_CP = pltpu.CompilerParams(needs_layout_passes=False)
_NW = 32
_CHUNK = _N // 4
_NVREG = _CHUNK // 16
_ROUNDS = ((19, 12), (7, 12), (0, 7))


def _raw_ce(L, tgt, full):
    l0, l1, l2 = L[0], L[1], L[2]
    if full:
        m = jnp.maximum(jnp.maximum(l0, l1), l2)
        lse = jnp.log(jnp.exp(l0 - m) + jnp.exp(l1 - m) + jnp.exp(l2 - m)) + m
        lt = jnp.where(tgt == 0, l0, jnp.where(tgt == 1, l1, l2))
    else:
        m = jnp.maximum(l0, l1)
        lse = jnp.log(jnp.exp(l0 - m) + jnp.exp(l1 - m)) + m
        lt = jnp.where(tgt >= 1, l1, l0)
    return jnp.maximum(lse - lt, 0.0)


def _ce_kernel(sel_ref, cls_ref, l1_ref, l2_ref, rawA_ref, rawB_ref, tot_ref):
    j = pl.program_id(0)
    sel_full = sel_ref[j, 1] > 0.5
    for i in (1, 2):
        L = l1_ref[0] if i == 1 else l2_ref[0]
        tgt = cls_ref[0, i]
        raw = lax.cond(sel_full,
                       lambda L=L, tgt=tgt: _raw_ce(L, tgt, True),
                       lambda L=L, tgt=tgt: _raw_ce(L, tgt, False))
        out_ref = rawA_ref if i == 1 else rawB_ref
        out_ref[0] = raw
        tot = jnp.sum(raw).reshape(1, 1)
        tot_ref[pl.ds((i - 1) * 4 + j, 1), :] = jnp.broadcast_to(tot, (1, 128))


def _run_ce(cls_gt, logits_1, logits_2, selector):
    return pl.pallas_call(
        _ce_kernel,
        grid=(4,),
        in_specs=[
            pl.BlockSpec(memory_space=pltpu.SMEM),
            pl.BlockSpec((1, 3, _H, _W), lambda j: (j, 0, 0, 0)),
            pl.BlockSpec((1, 3, _H, _W), lambda j: (j, 0, 0, 0)),
            pl.BlockSpec((1, 3, _H, _W), lambda j: (j, 0, 0, 0)),
        ],
        out_specs=[
            pl.BlockSpec((1, _H, _W), lambda j: (j, 0, 0)),
            pl.BlockSpec((1, _H, _W), lambda j: (j, 0, 0)),
            pl.BlockSpec((8, 128), lambda j: (0, 0)),
        ],
        out_shape=[
            jax.ShapeDtypeStruct((4, _H, _W), jnp.float32),
            jax.ShapeDtypeStruct((4, _H, _W), jnp.float32),
            jax.ShapeDtypeStruct((8, 128), jnp.float32),
        ],
    )(selector, cls_gt, logits_1, logits_2)


def _lane0(vec):
    return jnp.sum(jnp.where(lax.iota(jnp.int32, 16) == 0, vec, 0))


def _lane1(vec):
    return jnp.sum(jnp.where(lax.iota(jnp.int32, 16) == 1, vec, 0))


_LS = 4


def _make_hist(shift, width, first):
    nb = 1 << width

    @functools.partial(
        pl.kernel,
        out_type=[jax.ShapeDtypeStruct((_NW, nb), jnp.int32),
                  jax.ShapeDtypeStruct((_NW, nb), jnp.float32)],
        mesh=_MESH, compiler_params=_CP,
        scratch_types=[pltpu.VMEM((_CHUNK,), jnp.float32),
                       pltpu.VMEM((_LS * nb,), jnp.int32),
                       pltpu.VMEM((_LS * nb,), jnp.float32),
                       pltpu.VMEM((16,), jnp.int32)])
    def hist_kernel(rawA, rawB, state_i, hist_i, hist_f, data_v, hi_v, hf_v,
                    st_v):
        c = lax.axis_index("c")
        s = lax.axis_index("s")
        wid = c * 16 + s
        p = wid // 4
        q = wid % 4

        @pl.when(p < 4)
        def _():
            pltpu.sync_copy(rawA.at[p, pl.ds(q * _CHUNK, _CHUNK)], data_v)

        @pl.when(p >= 4)
        def _():
            pltpu.sync_copy(rawB.at[p - 4, pl.ds(q * _CHUNK, _CHUNK)], data_v)

        zi = jnp.zeros((16,), jnp.int32)
        zf = jnp.zeros((16,), jnp.float32)

        def zero(i, _):
            hi_v[pl.ds(i * 16, 16)] = zi
            hf_v[pl.ds(i * 16, 16)] = zf
            return 0
        lax.fori_loop(0, _LS * nb // 16, zero, 0)

        if first:
            prefix = jnp.int32(0)
        else:
            pltpu.sync_copy(state_i.at[p], st_v)
            prefix = _lane0(st_v[...])

        ones_i = jnp.ones((16,), jnp.int32)
        lanebank = (lax.iota(jnp.int32, 16) & (_LS - 1)) * nb

        def body(i, _):
            base = i * 64
            for k in range(4):
                x = data_v[pl.ds(base + k * 16, 16)]
                u = plsc.bitcast(x, jnp.int32)
                d = (lax.shift_right_logical(u, shift) & (nb - 1)) + lanebank
                if first:
                    plsc.addupdate_scatter(hi_v, [d], ones_i)
                    plsc.addupdate_scatter(hf_v, [d], x)
                else:
                    m = lax.shift_right_logical(u, shift + width) == prefix
                    plsc.addupdate_scatter(hi_v, [d], ones_i, mask=m)
                    plsc.addupdate_scatter(hf_v, [d], x, mask=m)
            return 0
        lax.fori_loop(0, _NVREG // 4, body, 0)

        def fold(j, _):
            sl = pl.ds(j * 16, 16)
            acc_i = hi_v[sl]
            acc_f = hf_v[sl]
            for b in range(1, _LS):
                slb = pl.ds(b * nb + j * 16, 16)
                acc_i = acc_i + hi_v[slb]
                acc_f = acc_f + hf_v[slb]
            hi_v[sl] = acc_i
            hf_v[sl] = acc_f
            return 0
        lax.fori_loop(0, nb // 16, fold, 0)

        pltpu.sync_copy(hi_v.at[pl.ds(0, nb)], hist_i.at[wid])
        pltpu.sync_copy(hf_v.at[pl.ds(0, nb)], hist_f.at[wid])

    return hist_kernel


def _make_merge(shift, width, first, last):
    nb = 1 << width
    nrow = nb // 16

    @functools.partial(
        pl.kernel,
        out_type=[jax.ShapeDtypeStruct((8, 16), jnp.int32),
                  jax.ShapeDtypeStruct((8, 16), jnp.float32),
                  jax.ShapeDtypeStruct((8, 16), jnp.float32)],
        mesh=_MESH, compiler_params=_CP,
        scratch_types=[pltpu.VMEM((4, nb), jnp.int32),
                       pltpu.VMEM((4, nb), jnp.float32),
                       pltpu.VMEM((16,), jnp.int32),
                       pltpu.VMEM((16,), jnp.float32)])
    def merge_kernel(hist_i, hist_f, state_i, state_f, out_i, out_f, out_tk,
                     h4_v, f4_v, ti_v, tf_v):
        c = lax.axis_index("c")
        s = lax.axis_index("s")

        @pl.when(s < 4)
        def _():
            p = s * 2 + c
            pltpu.sync_copy(hist_i.at[pl.ds(p * 4, 4)], h4_v)
            pltpu.sync_copy(hist_f.at[pl.ds(p * 4, 4)], f4_v)

            if first:
                prefix = jnp.int32(0)
                c_above = jnp.int32(0)
                s_above = jnp.float32(0.0)
            else:
                pltpu.sync_copy(state_i.at[p], ti_v)
                pltpu.sync_copy(state_f.at[p], tf_v)
                prefix = _lane0(ti_v[...])
                c_above = _lane1(ti_v[...])
                s_above = jnp.sum(
                    jnp.where(lax.iota(jnp.int32, 16) == 0, tf_v[...], 0.0))

            kk = jnp.int32(_K) - c_above
            iota = lax.iota(jnp.int32, 16)

            def row_cf(j):
                sl = pl.ds(j * 16, 16)
                rowc = h4_v[0, sl] + h4_v[1, sl] + h4_v[2, sl] + h4_v[3, sl]
                rowf = f4_v[0, sl] + f4_v[1, sl] + f4_v[2, sl] + f4_v[3, sl]
                return rowc, rowf

            def scan(t, carry):
                run_c, run_f, found, jc, cb, sb = carry
                j = nrow - 1 - t
                rowc, rowf = row_cf(j)
                rc = jnp.sum(rowc)
                rf = jnp.sum(rowf)
                take = jnp.logical_and(found == 0, run_c + rc >= kk)
                jc = jnp.where(take, j, jc)
                cb = jnp.where(take, run_c, cb)
                sb = jnp.where(take, run_f, sb)
                found = jnp.where(take, jnp.int32(1), found)
                return run_c + rc, run_f + rf, found, jc, cb, sb

            init = (jnp.int32(0), jnp.float32(0.0), jnp.int32(0),
                    jnp.int32(0), jnp.int32(0), jnp.float32(0.0))
            _, _, _, jc, cb, sb = lax.fori_loop(0, nrow, scan, init)

            rowc, rowf = row_cf(jc)
            rev_c = lax.rev(rowc, (0,))
            rev_f = lax.rev(rowf, (0,))
            cum_c = plsc.cumsum(rev_c)
            cum_f = plsc.cumsum(rev_f)
            crossed = (cb + cum_c) >= kk
            lstar = plsc.all_reduce_ffs(crossed)
            l0 = jnp.sum(jnp.where(iota == 0, lstar, 0))
            csel = jnp.sum(jnp.where(iota == l0, cum_c, 0))
            fsel = jnp.sum(jnp.where(iota == l0, cum_f, 0))
            crev = jnp.sum(jnp.where(iota == l0, rev_c, 0))
            frev = jnp.sum(jnp.where(iota == l0, rev_f, 0))
            digit = jc * 16 + 15 - l0
            c_inc = cb + csel - crev
            s_inc = sb + fsel - frev

            new_prefix = (prefix << width) | digit
            new_c = c_above + c_inc
            new_s = s_above + s_inc

            ti_v[...] = jnp.where(iota == 0, new_prefix,
                                  jnp.where(iota == 1, new_c, 0))
            pltpu.sync_copy(ti_v, out_i.at[p])
            tf_v[...] = jnp.where(iota == 0, new_s, 0.0)
            pltpu.sync_copy(tf_v, out_f.at[p])

            if last:
                v = plsc.bitcast(jnp.zeros((16,), jnp.int32) + new_prefix,
                                 jnp.float32)
                tk = new_s + (jnp.float32(_K) - new_c.astype(jnp.float32)) * v
                tf_v[...] = jnp.zeros((16,), jnp.float32) + tk
                pltpu.sync_copy(tf_v, out_tk.at[p])

    return merge_kernel


_HIST = [_make_hist(sh, w, r == 0) for r, (sh, w) in enumerate(_ROUNDS)]
_MERGE = [_make_merge(sh, w, r == 0, r == len(_ROUNDS) - 1)
          for r, (sh, w) in enumerate(_ROUNDS)]


def _sc_topk_sum(rawA, rawB):
    state_i = jnp.zeros((8, 16), jnp.int32)
    state_f = jnp.zeros((8, 16), jnp.float32)
    tk = None
    for r in range(len(_ROUNDS)):
        hist_i, hist_f = _HIST[r](rawA, rawB, state_i)
        state_i, state_f, tk = _MERGE[r](hist_i, hist_f, state_i, state_f)
    return tk


def kernel(gt, cls_gt, logits_1, logits_2, selector, it):
    rawA, rawB, tot = _run_ce(cls_gt, logits_1, logits_2, selector)
    tk = _sc_topk_sum(rawA.reshape(4, _N), rawB.reshape(4, _N))
    per_topk = tk[:, 0] / jnp.float32(_K)
    per_tot = tot[:, 0] / jnp.float32(_N)
    per = jnp.where(it < _START_WARM, per_tot, per_topk)
    return jnp.sum(per) * jnp.float32(0.25)


# speedup vs baseline: 1.0917x; 1.0917x over previous
import functools

import jax
import jax.numpy as jnp
from jax import lax
from jax.experimental import pallas as pl
from jax.experimental.pallas import tpu as pltpu
from jax.experimental.pallas import tpu_sc as plsc

_H = 384
_W = 384
_N = _H * _W
_K = int(_N * 0.15)
_START_WARM = 20000

_MESH = plsc.VectorSubcoreMesh(core_axis_name="c", subcore_axis_name="s")
_CP = pltpu.CompilerParams(needs_layout_passes=False)
_NW = 32
_CHUNK = _N // 4
_NVREG = _CHUNK // 16
_ROUNDS = ((19, 12), (7, 12), (0, 7))


def _raw_ce(L, tgt, full):
    l0, l1, l2 = L[0], L[1], L[2]
    if full:
        m = jnp.maximum(jnp.maximum(l0, l1), l2)
        lse = jnp.log(jnp.exp(l0 - m) + jnp.exp(l1 - m) + jnp.exp(l2 - m)) + m
        lt = jnp.where(tgt == 0, l0, jnp.where(tgt == 1, l1, l2))
    else:
        m = jnp.maximum(l0, l1)
        lse = jnp.log(jnp.exp(l0 - m) + jnp.exp(l1 - m)) + m
        lt = jnp.where(tgt >= 1, l1, l0)
    return jnp.maximum(lse - lt, 0.0)


def _ce_kernel(sel_ref, cls_ref, l1_ref, l2_ref, rawA_ref, rawB_ref, tot_ref):
    j = pl.program_id(0)
    sel_full = sel_ref[j, 1] > 0.5
    for i in (1, 2):
        L = l1_ref[0] if i == 1 else l2_ref[0]
        tgt = cls_ref[0, i]
        raw = lax.cond(sel_full,
                       lambda L=L, tgt=tgt: _raw_ce(L, tgt, True),
                       lambda L=L, tgt=tgt: _raw_ce(L, tgt, False))
        out_ref = rawA_ref if i == 1 else rawB_ref
        out_ref[0] = raw
        tot = jnp.sum(raw).reshape(1, 1)
        tot_ref[pl.ds((i - 1) * 4 + j, 1), :] = jnp.broadcast_to(tot, (1, 128))


def _run_ce(cls_gt, logits_1, logits_2, selector):
    return pl.pallas_call(
        _ce_kernel,
        grid=(4,),
        in_specs=[
            pl.BlockSpec(memory_space=pltpu.SMEM),
            pl.BlockSpec((1, 3, _H, _W), lambda j: (j, 0, 0, 0)),
            pl.BlockSpec((1, 3, _H, _W), lambda j: (j, 0, 0, 0)),
            pl.BlockSpec((1, 3, _H, _W), lambda j: (j, 0, 0, 0)),
        ],
        out_specs=[
            pl.BlockSpec((1, _H, _W), lambda j: (j, 0, 0)),
            pl.BlockSpec((1, _H, _W), lambda j: (j, 0, 0)),
            pl.BlockSpec((8, 128), lambda j: (0, 0)),
        ],
        out_shape=[
            jax.ShapeDtypeStruct((4, _H, _W), jnp.float32),
            jax.ShapeDtypeStruct((4, _H, _W), jnp.float32),
            jax.ShapeDtypeStruct((8, 128), jnp.float32),
        ],
    )(selector, cls_gt, logits_1, logits_2)


def _lane0(vec):
    return jnp.sum(jnp.where(lax.iota(jnp.int32, 16) == 0, vec, 0))


def _lane1(vec):
    return jnp.sum(jnp.where(lax.iota(jnp.int32, 16) == 1, vec, 0))


def _make_hist(shift, width, first):
    nb = 1 << width

    @functools.partial(
        pl.kernel,
        out_type=jax.ShapeDtypeStruct((_NW, nb), jnp.int32),
        mesh=_MESH, compiler_params=_CP,
        scratch_types=[pltpu.VMEM((_CHUNK,), jnp.float32),
                       pltpu.VMEM((nb,), jnp.int32),
                       pltpu.VMEM((16,), jnp.int32)])
    def hist_kernel(rawA, rawB, state_i, hist_i, data_v, hi_v, st_v):
        c = lax.axis_index("c")
        s = lax.axis_index("s")
        wid = c * 16 + s
        p = wid // 4
        q = wid % 4

        @pl.when(p < 4)
        def _():
            pltpu.sync_copy(rawA.at[p, pl.ds(q * _CHUNK, _CHUNK)], data_v)

        @pl.when(p >= 4)
        def _():
            pltpu.sync_copy(rawB.at[p - 4, pl.ds(q * _CHUNK, _CHUNK)], data_v)

        zi = jnp.zeros((16,), jnp.int32)

        def zero(i, _):
            hi_v[pl.ds(i * 16, 16)] = zi
            return 0
        lax.fori_loop(0, nb // 16, zero, 0)

        if first:
            prefix = jnp.int32(0)
        else:
            pltpu.sync_copy(state_i.at[p], st_v)
            prefix = _lane0(st_v[...])

        ones_i = jnp.ones((16,), jnp.int32)

        def body(i, _):
            base = i * 64
            for k in range(4):
                x = data_v[pl.ds(base + k * 16, 16)]
                u = plsc.bitcast(x, jnp.int32)
                d = lax.shift_right_logical(u, shift) & (nb - 1)
                if first:
                    plsc.addupdate_scatter(hi_v, [d], ones_i)
                else:
                    m = lax.shift_right_logical(u, shift + width) == prefix
                    plsc.addupdate_scatter(hi_v, [d], ones_i, mask=m)
            return 0
        lax.fori_loop(0, _NVREG // 4, body, 0)

        pltpu.sync_copy(hi_v, hist_i.at[wid])

    return hist_kernel


def _make_merge(shift, width, first):
    nb = 1 << width
    nrow = nb // 16

    @functools.partial(
        pl.kernel,
        out_type=jax.ShapeDtypeStruct((8, 16), jnp.int32),
        mesh=_MESH, compiler_params=_CP,
        scratch_types=[pltpu.VMEM((4, nb), jnp.int32),
                       pltpu.VMEM((16,), jnp.int32)])
    def merge_kernel(hist_i, state_i, out_i, h4_v, ti_v):
        c = lax.axis_index("c")
        s = lax.axis_index("s")

        @pl.when(s < 4)
        def _():
            p = s * 2 + c
            pltpu.sync_copy(hist_i.at[pl.ds(p * 4, 4)], h4_v)

            if first:
                prefix = jnp.int32(0)
                c_above = jnp.int32(0)
            else:
                pltpu.sync_copy(state_i.at[p], ti_v)
                prefix = _lane0(ti_v[...])
                c_above = _lane1(ti_v[...])

            kk = jnp.int32(_K) - c_above
            iota = lax.iota(jnp.int32, 16)

            def row_c(j):
                sl = pl.ds(j * 16, 16)
                return h4_v[0, sl] + h4_v[1, sl] + h4_v[2, sl] + h4_v[3, sl]

            def scan(t, carry):
                run_c, found, jc, cb = carry
                j = nrow - 1 - t
                rc = jnp.sum(row_c(j))
                take = jnp.logical_and(found == 0, run_c + rc >= kk)
                jc = jnp.where(take, j, jc)
                cb = jnp.where(take, run_c, cb)
                found = jnp.where(take, jnp.int32(1), found)
                return run_c + rc, found, jc, cb

            init = (jnp.int32(0), jnp.int32(0), jnp.int32(0), jnp.int32(0))
            _, _, jc, cb = lax.fori_loop(0, nrow, scan, init)

            rowc = row_c(jc)
            rev_c = lax.rev(rowc, (0,))
            cum_c = plsc.cumsum(rev_c)
            crossed = (cb + cum_c) >= kk
            lstar = plsc.all_reduce_ffs(crossed)
            l0 = jnp.sum(jnp.where(iota == 0, lstar, 0))
            csel = jnp.sum(jnp.where(iota == l0, cum_c, 0))
            crev = jnp.sum(jnp.where(iota == l0, rev_c, 0))
            digit = jc * 16 + 15 - l0
            c_inc = cb + csel - crev

            new_prefix = (prefix << width) | digit
            new_c = c_above + c_inc

            ti_v[...] = jnp.where(iota == 0, new_prefix,
                                  jnp.where(iota == 1, new_c, 0))
            pltpu.sync_copy(ti_v, out_i.at[p])

    return merge_kernel


@functools.partial(
    pl.kernel,
    out_type=jax.ShapeDtypeStruct((_NW, 16), jnp.float32),
    mesh=_MESH, compiler_params=_CP,
    scratch_types=[pltpu.VMEM((_CHUNK,), jnp.float32),
                   pltpu.VMEM((16,), jnp.int32),
                   pltpu.VMEM((16,), jnp.float32)])
def _sum_above_kernel(rawA, rawB, state_i, out_f, data_v, st_v, tf_v):
    c = lax.axis_index("c")
    s = lax.axis_index("s")
    wid = c * 16 + s
    p = wid // 4
    q = wid % 4

    @pl.when(p < 4)
    def _():
        pltpu.sync_copy(rawA.at[p, pl.ds(q * _CHUNK, _CHUNK)], data_v)

    @pl.when(p >= 4)
    def _():
        pltpu.sync_copy(rawB.at[p - 4, pl.ds(q * _CHUNK, _CHUNK)], data_v)

    pltpu.sync_copy(state_i.at[p], st_v)
    vbits = _lane0(st_v[...])
    v = plsc.bitcast(jnp.zeros((16,), jnp.int32) + vbits, jnp.float32)
    zf = jnp.zeros((16,), jnp.float32)

    def body(i, acc):
        base = i * 64
        for k in range(4):
            x = data_v[pl.ds(base + k * 16, 16)]
            acc = acc + jnp.where(x > v, x, zf)
        return acc
    acc = lax.fori_loop(0, _NVREG // 4, body, zf)
    tf_v[...] = acc
    pltpu.sync_copy(tf_v, out_f.at[wid])


_HIST = [_make_hist(sh, w, r == 0) for r, (sh, w) in enumerate(_ROUNDS)]
_MERGE = [_make_merge(sh, w, r == 0) for r, (sh, w) in enumerate(_ROUNDS)]


def kernel(gt, cls_gt, logits_1, logits_2, selector, it):
    rawA, rawB, tot = _run_ce(cls_gt, logits_1, logits_2, selector)
    rawA = rawA.reshape(4, _N)
    rawB = rawB.reshape(4, _N)
    state_i = jnp.zeros((8, 16), jnp.int32)
    for r in range(len(_ROUNDS)):
        hist_i = _HIST[r](rawA, rawB, state_i)
        state_i = _MERGE[r](hist_i, state_i)
    parts = _sum_above_kernel(rawA, rawB, state_i)
    v = lax.bitcast_convert_type(state_i[:, 0], jnp.float32)
    c_above = state_i[:, 1].astype(jnp.float32)
    s_above = jnp.sum(parts.reshape(8, 4 * 16), axis=1)
    tk = s_above + (jnp.float32(_K) - c_above) * v
    per_topk = tk / jnp.float32(_K)
    per_tot = tot[:, 0] / jnp.float32(_N)
    per = jnp.where(it < _START_WARM, per_tot, per_topk)
    return jnp.sum(per) * jnp.float32(0.25)


# speedup vs baseline: 1.6701x; 1.5298x over previous
import functools

import jax
import jax.numpy as jnp
from jax import lax
from jax.experimental import pallas as pl
from jax.experimental.pallas import tpu as pltpu
from jax.experimental.pallas import tpu_sc as plsc

_H = 384
_W = 384
_N = _H * _W
_K = int(_N * 0.15)
_START_WARM = 20000

_MESH = plsc.VectorSubcoreMesh(core_axis_name="c", subcore_axis_name="s")
_CP = pltpu.CompilerParams(needs_layout_passes=False)
_NW = 32
_CHUNK = _N // 4
_NVREG = _CHUNK // 16
_ROUNDS = ((19, 12), (7, 12), (0, 7))


def _raw_ce(L, tgt, full):
    l0, l1, l2 = L[0], L[1], L[2]
    if full:
        m = jnp.maximum(jnp.maximum(l0, l1), l2)
        lse = jnp.log(jnp.exp(l0 - m) + jnp.exp(l1 - m) + jnp.exp(l2 - m)) + m
        lt = jnp.where(tgt == 0, l0, jnp.where(tgt == 1, l1, l2))
    else:
        m = jnp.maximum(l0, l1)
        lse = jnp.log(jnp.exp(l0 - m) + jnp.exp(l1 - m)) + m
        lt = jnp.where(tgt >= 1, l1, l0)
    return jnp.maximum(lse - lt, 0.0)


def _ce_kernel(sel_ref, cls_ref, l1_ref, l2_ref, rawA_ref, rawB_ref, tot_ref):
    j = pl.program_id(0)
    sel_full = sel_ref[j, 1] > 0.5
    for i in (1, 2):
        L = l1_ref[0] if i == 1 else l2_ref[0]
        tgt = cls_ref[0, i]
        raw = lax.cond(sel_full,
                       lambda L=L, tgt=tgt: _raw_ce(L, tgt, True),
                       lambda L=L, tgt=tgt: _raw_ce(L, tgt, False))
        out_ref = rawA_ref if i == 1 else rawB_ref
        out_ref[0] = raw
        tot = jnp.sum(raw).reshape(1, 1)
        tot_ref[pl.ds((i - 1) * 4 + j, 1), :] = jnp.broadcast_to(tot, (1, 128))


def _run_ce(cls_gt, logits_1, logits_2, selector):
    return pl.pallas_call(
        _ce_kernel,
        grid=(4,),
        in_specs=[
            pl.BlockSpec(memory_space=pltpu.SMEM),
            pl.BlockSpec((1, 3, _H, _W), lambda j: (j, 0, 0, 0)),
            pl.BlockSpec((1, 3, _H, _W), lambda j: (j, 0, 0, 0)),
            pl.BlockSpec((1, 3, _H, _W), lambda j: (j, 0, 0, 0)),
        ],
        out_specs=[
            pl.BlockSpec((1, _H, _W), lambda j: (j, 0, 0)),
            pl.BlockSpec((1, _H, _W), lambda j: (j, 0, 0)),
            pl.BlockSpec((8, 128), lambda j: (0, 0)),
        ],
        out_shape=[
            jax.ShapeDtypeStruct((4, _H, _W), jnp.float32),
            jax.ShapeDtypeStruct((4, _H, _W), jnp.float32),
            jax.ShapeDtypeStruct((8, 128), jnp.float32),
        ],
    )(selector, cls_gt, logits_1, logits_2)


def _lane0(vec):
    return jnp.sum(jnp.where(lax.iota(jnp.int32, 16) == 0, vec, 0))


def _lane1(vec):
    return jnp.sum(jnp.where(lax.iota(jnp.int32, 16) == 1, vec, 0))


def _make_hist(shift, width, first):
    nb = 1 << width

    @functools.partial(
        pl.kernel,
        out_type=jax.ShapeDtypeStruct((_NW, nb), jnp.int32),
        mesh=_MESH, compiler_params=_CP,
        scratch_types=[pltpu.VMEM((_CHUNK,), jnp.float32),
                       pltpu.VMEM((nb,), jnp.int32),
                       pltpu.VMEM((16,), jnp.int32)])
    def hist_kernel(rawA, rawB, state_i, hist_i, data_v, hi_v, st_v):
        c = lax.axis_index("c")
        s = lax.axis_index("s")
        wid = c * 16 + s
        p = wid // 4
        q = wid % 4

        @pl.when(p < 4)
        def _():
            pltpu.sync_copy(rawA.at[p, pl.ds(q * _CHUNK, _CHUNK)], data_v)

        @pl.when(p >= 4)
        def _():
            pltpu.sync_copy(rawB.at[p - 4, pl.ds(q * _CHUNK, _CHUNK)], data_v)

        zi = jnp.zeros((16,), jnp.int32)

        @plsc.parallel_loop(0, nb // 16, 1, unroll=8)
        def _zero(i):
            hi_v[pl.ds(i * 16, 16)] = zi

        if first:
            prefix = jnp.int32(0)
        else:
            pltpu.sync_copy(state_i.at[p], st_v)
            prefix = _lane0(st_v[...])

        ones_i = jnp.ones((16,), jnp.int32)

        @plsc.parallel_loop(0, _NVREG, 1, unroll=8)
        def _body(i):
            x = data_v[pl.ds(i * 16, 16)]
            u = plsc.bitcast(x, jnp.int32)
            if first:
                d = lax.shift_right_logical(u, shift)
                plsc.addupdate_scatter(hi_v, [d], ones_i)
            else:
                d = lax.shift_right_logical(u, shift) & (nb - 1)
                m = lax.shift_right_logical(u, shift + width) == prefix
                plsc.addupdate_scatter(hi_v, [d], ones_i, mask=m)

        pltpu.sync_copy(hi_v, hist_i.at[wid])

    return hist_kernel


def _make_merge(shift, width, first):
    nb = 1 << width
    nrow = nb // 16

    @functools.partial(
        pl.kernel,
        out_type=jax.ShapeDtypeStruct((8, 16), jnp.int32),
        mesh=_MESH, compiler_params=_CP,
        scratch_types=[pltpu.VMEM((4, nb), jnp.int32),
                       pltpu.VMEM((16,), jnp.int32)])
    def merge_kernel(hist_i, state_i, out_i, h4_v, ti_v):
        c = lax.axis_index("c")
        s = lax.axis_index("s")

        @pl.when(s < 4)
        def _():
            p = s * 2 + c
            pltpu.sync_copy(hist_i.at[pl.ds(p * 4, 4)], h4_v)

            if first:
                prefix = jnp.int32(0)
                c_above = jnp.int32(0)
            else:
                pltpu.sync_copy(state_i.at[p], ti_v)
                prefix = _lane0(ti_v[...])
                c_above = _lane1(ti_v[...])

            kk = jnp.int32(_K) - c_above
            iota = lax.iota(jnp.int32, 16)

            def row_c(j):
                sl = pl.ds(j * 16, 16)
                return h4_v[0, sl] + h4_v[1, sl] + h4_v[2, sl] + h4_v[3, sl]

            def scan(t, carry):
                run_c, found, jc, cb = carry
                j = nrow - 1 - t
                rc = jnp.sum(row_c(j))
                take = jnp.logical_and(found == 0, run_c + rc >= kk)
                jc = jnp.where(take, j, jc)
                cb = jnp.where(take, run_c, cb)
                found = jnp.where(take, jnp.int32(1), found)
                return run_c + rc, found, jc, cb

            init = (jnp.int32(0), jnp.int32(0), jnp.int32(0), jnp.int32(0))
            _, _, jc, cb = lax.fori_loop(0, nrow, scan, init)

            rowc = row_c(jc)
            rev_c = lax.rev(rowc, (0,))
            cum_c = plsc.cumsum(rev_c)
            crossed = (cb + cum_c) >= kk
            lstar = plsc.all_reduce_ffs(crossed)
            l0 = jnp.sum(jnp.where(iota == 0, lstar, 0))
            csel = jnp.sum(jnp.where(iota == l0, cum_c, 0))
            crev = jnp.sum(jnp.where(iota == l0, rev_c, 0))
            digit = jc * 16 + 15 - l0
            c_inc = cb + csel - crev

            new_prefix = (prefix << width) | digit
            new_c = c_above + c_inc

            ti_v[...] = jnp.where(iota == 0, new_prefix,
                                  jnp.where(iota == 1, new_c, 0))
            pltpu.sync_copy(ti_v, out_i.at[p])

    return merge_kernel


@functools.partial(
    pl.kernel,
    out_type=jax.ShapeDtypeStruct((_NW, 16), jnp.float32),
    mesh=_MESH, compiler_params=_CP,
    scratch_types=[pltpu.VMEM((_CHUNK,), jnp.float32),
                   pltpu.VMEM((16,), jnp.int32),
                   pltpu.VMEM((16,), jnp.float32)])
def _sum_above_kernel(rawA, rawB, state_i, out_f, data_v, st_v, tf_v):
    c = lax.axis_index("c")
    s = lax.axis_index("s")
    wid = c * 16 + s
    p = wid // 4
    q = wid % 4

    @pl.when(p < 4)
    def _():
        pltpu.sync_copy(rawA.at[p, pl.ds(q * _CHUNK, _CHUNK)], data_v)

    @pl.when(p >= 4)
    def _():
        pltpu.sync_copy(rawB.at[p - 4, pl.ds(q * _CHUNK, _CHUNK)], data_v)

    pltpu.sync_copy(state_i.at[p], st_v)
    vbits = _lane0(st_v[...])
    v = plsc.bitcast(jnp.zeros((16,), jnp.int32) + vbits, jnp.float32)
    zf = jnp.zeros((16,), jnp.float32)

    @plsc.parallel_loop(0, _NVREG // 4, 1, unroll=2, carry=(zf, zf, zf, zf))
    def _accs(i, accs):
        a0, a1, a2, a3 = accs
        base = i * 64
        x0 = data_v[pl.ds(base, 16)]
        x1 = data_v[pl.ds(base + 16, 16)]
        x2 = data_v[pl.ds(base + 32, 16)]
        x3 = data_v[pl.ds(base + 48, 16)]
        return (a0 + jnp.where(x0 > v, x0, zf),
                a1 + jnp.where(x1 > v, x1, zf),
                a2 + jnp.where(x2 > v, x2, zf),
                a3 + jnp.where(x3 > v, x3, zf))
    a0, a1, a2, a3 = _accs
    tf_v[...] = (a0 + a1) + (a2 + a3)
    pltpu.sync_copy(tf_v, out_f.at[wid])


_HIST = [_make_hist(sh, w, r == 0) for r, (sh, w) in enumerate(_ROUNDS)]
_MERGE = [_make_merge(sh, w, r == 0) for r, (sh, w) in enumerate(_ROUNDS)]


def kernel(gt, cls_gt, logits_1, logits_2, selector, it):
    rawA, rawB, tot = _run_ce(cls_gt, logits_1, logits_2, selector)
    rawA = rawA.reshape(4, _N)
    rawB = rawB.reshape(4, _N)
    state_i = jnp.zeros((8, 16), jnp.int32)
    for r in range(len(_ROUNDS)):
        hist_i = _HIST[r](rawA, rawB, state_i)
        state_i = _MERGE[r](hist_i, state_i)
    parts = _sum_above_kernel(rawA, rawB, state_i)
    v = lax.bitcast_convert_type(state_i[:, 0], jnp.float32)
    c_above = state_i[:, 1].astype(jnp.float32)
    s_above = jnp.sum(parts.reshape(8, 4 * 16), axis=1)
    tk = s_above + (jnp.float32(_K) - c_above) * v
    per_topk = tk / jnp.float32(_K)
    per_tot = tot[:, 0] / jnp.float32(_N)
    per = jnp.where(it < _START_WARM, per_tot, per_topk)
    return jnp.sum(per) * jnp.float32(0.25)


# speedup vs baseline: 1.8814x; 1.1265x over previous
import functools

import jax
import jax.numpy as jnp
from jax import lax
from jax.experimental import pallas as pl
from jax.experimental.pallas import tpu as pltpu
from jax.experimental.pallas import tpu_sc as plsc

_H = 384
_W = 384
_N = _H * _W
_K = int(_N * 0.15)
_START_WARM = 20000

_MESH = plsc.VectorSubcoreMesh(core_axis_name="c", subcore_axis_name="s")
_CP = pltpu.CompilerParams(needs_layout_passes=False)
_NW = 32
_CHUNK = _N // 4
_NVREG = _CHUNK // 16
_ROUNDS = ((19, 12), (7, 12), (0, 7))


def _raw_ce(L, tgt, full):
    l0, l1, l2 = L[0], L[1], L[2]
    if full:
        m = jnp.maximum(jnp.maximum(l0, l1), l2)
        lse = jnp.log(jnp.exp(l0 - m) + jnp.exp(l1 - m) + jnp.exp(l2 - m)) + m
        lt = jnp.where(tgt == 0, l0, jnp.where(tgt == 1, l1, l2))
    else:
        m = jnp.maximum(l0, l1)
        lse = jnp.log(jnp.exp(l0 - m) + jnp.exp(l1 - m)) + m
        lt = jnp.where(tgt >= 1, l1, l0)
    return jnp.maximum(lse - lt, 0.0)


def _ce_kernel(sel_ref, cls_ref, l1_ref, l2_ref, rawA_ref, rawB_ref, tot_ref):
    j = pl.program_id(0)
    sel_full = sel_ref[j, 1] > 0.5
    for i in (1, 2):
        L = l1_ref[0] if i == 1 else l2_ref[0]
        tgt = cls_ref[0, i]
        raw = lax.cond(sel_full,
                       lambda L=L, tgt=tgt: _raw_ce(L, tgt, True),
                       lambda L=L, tgt=tgt: _raw_ce(L, tgt, False))
        out_ref = rawA_ref if i == 1 else rawB_ref
        out_ref[0] = raw
        tot = jnp.sum(raw).reshape(1, 1)
        tot_ref[pl.ds((i - 1) * 4 + j, 1), :] = jnp.broadcast_to(tot, (1, 128))


def _run_ce(cls_gt, logits_1, logits_2, selector):
    return pl.pallas_call(
        _ce_kernel,
        grid=(4,),
        in_specs=[
            pl.BlockSpec(memory_space=pltpu.SMEM),
            pl.BlockSpec((1, 3, _H, _W), lambda j: (j, 0, 0, 0)),
            pl.BlockSpec((1, 3, _H, _W), lambda j: (j, 0, 0, 0)),
            pl.BlockSpec((1, 3, _H, _W), lambda j: (j, 0, 0, 0)),
        ],
        out_specs=[
            pl.BlockSpec((1, _H, _W), lambda j: (j, 0, 0)),
            pl.BlockSpec((1, _H, _W), lambda j: (j, 0, 0)),
            pl.BlockSpec((8, 128), lambda j: (0, 0)),
        ],
        out_shape=[
            jax.ShapeDtypeStruct((4, _H, _W), jnp.float32),
            jax.ShapeDtypeStruct((4, _H, _W), jnp.float32),
            jax.ShapeDtypeStruct((8, 128), jnp.float32),
        ],
    )(selector, cls_gt, logits_1, logits_2)


def _lane0(vec):
    return jnp.sum(jnp.where(lax.iota(jnp.int32, 16) == 0, vec, 0))


def _lane1(vec):
    return jnp.sum(jnp.where(lax.iota(jnp.int32, 16) == 1, vec, 0))


def _scan_hist(h4_v, kk, nrow):
    iota = lax.iota(jnp.int32, 16)

    def row_c(j):
        sl = pl.ds(j * 16, 16)
        return h4_v[0, sl] + h4_v[1, sl] + h4_v[2, sl] + h4_v[3, sl]

    def scan(t, carry):
        run_c, found, jc, cb = carry
        j = nrow - 1 - t
        rc = jnp.sum(row_c(j))
        take = jnp.logical_and(found == 0, run_c + rc >= kk)
        jc = jnp.where(take, j, jc)
        cb = jnp.where(take, run_c, cb)
        found = jnp.where(take, jnp.int32(1), found)
        return run_c + rc, found, jc, cb

    init = (jnp.int32(0), jnp.int32(0), jnp.int32(0), jnp.int32(0))
    _, _, jc, cb = lax.fori_loop(0, nrow, scan, init)

    rowc = row_c(jc)
    rev_c = lax.rev(rowc, (0,))
    cum_c = plsc.cumsum(rev_c)
    crossed = (cb + cum_c) >= kk
    lstar = plsc.all_reduce_ffs(crossed)
    l0 = jnp.sum(jnp.where(iota == 0, lstar, 0))
    csel = jnp.sum(jnp.where(iota == l0, cum_c, 0))
    crev = jnp.sum(jnp.where(iota == l0, rev_c, 0))
    digit = jc * 16 + 15 - l0
    c_inc = cb + csel - crev
    return digit, c_inc


def _load_chunk(rawA, rawB, data_v, p, q):
    @pl.when(p < 4)
    def _():
        pltpu.sync_copy(rawA.at[p, pl.ds(q * _CHUNK, _CHUNK)], data_v)

    @pl.when(p >= 4)
    def _():
        pltpu.sync_copy(rawB.at[p - 4, pl.ds(q * _CHUNK, _CHUNK)], data_v)


@functools.partial(
    pl.kernel,
    out_type=jax.ShapeDtypeStruct((_NW, 4096), jnp.int32),
    mesh=_MESH, compiler_params=_CP,
    scratch_types=[pltpu.VMEM((_CHUNK,), jnp.float32),
                   pltpu.VMEM((4096,), jnp.int32)])
def _h0_kernel(rawA, rawB, hist_o, data_v, hi_v):
    c = lax.axis_index("c")
    s = lax.axis_index("s")
    wid = c * 16 + s
    _load_chunk(rawA, rawB, data_v, wid // 4, wid % 4)
    zi = jnp.zeros((16,), jnp.int32)

    @plsc.parallel_loop(0, 256, 1, unroll=8)
    def _zero(i):
        hi_v[pl.ds(i * 16, 16)] = zi

    ones_i = jnp.ones((16,), jnp.int32)

    @plsc.parallel_loop(0, _NVREG, 1, unroll=8)
    def _body(i):
        x = data_v[pl.ds(i * 16, 16)]
        u = plsc.bitcast(x, jnp.int32)
        d = lax.shift_right_logical(u, 19)
        plsc.addupdate_scatter(hi_v, [d], ones_i)

    pltpu.sync_copy(hi_v, hist_o.at[wid])


def _make_hist(r):
    shift, width = _ROUNDS[r]
    p_shift, p_width = _ROUNDS[r - 1]
    nb = 1 << width
    nb_prev = 1 << p_width
    first = r == 1

    @functools.partial(
        pl.kernel,
        out_type=[jax.ShapeDtypeStruct((_NW, nb), jnp.int32),
                  jax.ShapeDtypeStruct((8, 16), jnp.int32)],
        mesh=_MESH, compiler_params=_CP,
        scratch_types=[pltpu.VMEM((_CHUNK,), jnp.float32),
                       pltpu.VMEM((4, nb_prev), jnp.int32),
                       pltpu.VMEM((nb,), jnp.int32),
                       pltpu.VMEM((16,), jnp.int32)])
    def hist_kernel(rawA, rawB, prev_hist, prev_state, hist_o, state_o,
                    data_v, h4_v, hi_v, st_v):
        c = lax.axis_index("c")
        s = lax.axis_index("s")
        wid = c * 16 + s
        p = wid // 4
        q = wid % 4
        _load_chunk(rawA, rawB, data_v, p, q)
        pltpu.sync_copy(prev_hist.at[pl.ds(p * 4, 4)], h4_v)

        if first:
            prefix_prev = jnp.int32(0)
            c_prev = jnp.int32(0)
        else:
            pltpu.sync_copy(prev_state.at[p], st_v)
            prefix_prev = _lane0(st_v[...])
            c_prev = _lane1(st_v[...])

        digit, c_inc = _scan_hist(h4_v, jnp.int32(_K) - c_prev, nb_prev // 16)
        prefix = (prefix_prev << p_width) | digit
        c_above = c_prev + c_inc

        zi = jnp.zeros((16,), jnp.int32)

        @plsc.parallel_loop(0, nb // 16, 1, unroll=8)
        def _zero(i):
            hi_v[pl.ds(i * 16, 16)] = zi

        ones_i = jnp.ones((16,), jnp.int32)

        @plsc.parallel_loop(0, _NVREG, 1, unroll=8)
        def _body(i):
            x = data_v[pl.ds(i * 16, 16)]
            u = plsc.bitcast(x, jnp.int32)
            d = lax.shift_right_logical(u, shift) & (nb - 1)
            m = lax.shift_right_logical(u, shift + width) == prefix
            plsc.addupdate_scatter(hi_v, [d], ones_i, mask=m)

        pltpu.sync_copy(hi_v, hist_o.at[wid])

        @pl.when(q == 0)
        def _():
            iota = lax.iota(jnp.int32, 16)
            st_v[...] = jnp.where(iota == 0, prefix,
                                  jnp.where(iota == 1, c_above, 0))
            pltpu.sync_copy(st_v, state_o.at[p])

    return hist_kernel


@functools.partial(
    pl.kernel,
    out_type=[jax.ShapeDtypeStruct((_NW, 16), jnp.float32),
              jax.ShapeDtypeStruct((8, 16), jnp.int32)],
    mesh=_MESH, compiler_params=_CP,
    scratch_types=[pltpu.VMEM((_CHUNK,), jnp.float32),
                   pltpu.VMEM((4, 128), jnp.int32),
                   pltpu.VMEM((16,), jnp.int32),
                   pltpu.VMEM((16,), jnp.float32)])
def _sum_kernel(rawA, rawB, prev_hist, prev_state, parts_o, state_o,
                data_v, h4_v, st_v, tf_v):
    c = lax.axis_index("c")
    s = lax.axis_index("s")
    wid = c * 16 + s
    p = wid // 4
    q = wid % 4
    _load_chunk(rawA, rawB, data_v, p, q)
    pltpu.sync_copy(prev_hist.at[pl.ds(p * 4, 4)], h4_v)
    pltpu.sync_copy(prev_state.at[p], st_v)
    prefix_prev = _lane0(st_v[...])
    c_prev = _lane1(st_v[...])

    digit, c_inc = _scan_hist(h4_v, jnp.int32(_K) - c_prev, 8)
    vbits = (prefix_prev << _ROUNDS[2][1]) | digit
    c_above = c_prev + c_inc

    v = plsc.bitcast(jnp.zeros((16,), jnp.int32) + vbits, jnp.float32)
    zf = jnp.zeros((16,), jnp.float32)

    @plsc.parallel_loop(0, _NVREG // 4, 1, unroll=2, carry=(zf, zf, zf, zf))
    def _accs(i, accs):
        a0, a1, a2, a3 = accs
        base = i * 64
        x0 = data_v[pl.ds(base, 16)]
        x1 = data_v[pl.ds(base + 16, 16)]
        x2 = data_v[pl.ds(base + 32, 16)]
        x3 = data_v[pl.ds(base + 48, 16)]
        return (a0 + jnp.where(x0 > v, x0, zf),
                a1 + jnp.where(x1 > v, x1, zf),
                a2 + jnp.where(x2 > v, x2, zf),
                a3 + jnp.where(x3 > v, x3, zf))
    a0, a1, a2, a3 = _accs
    tf_v[...] = (a0 + a1) + (a2 + a3)
    pltpu.sync_copy(tf_v, parts_o.at[wid])

    @pl.when(q == 0)
    def _():
        iota = lax.iota(jnp.int32, 16)
        st_v[...] = jnp.where(iota == 0, vbits,
                              jnp.where(iota == 1, c_above, 0))
        pltpu.sync_copy(st_v, state_o.at[p])


_H1 = _make_hist(1)
_H2 = _make_hist(2)


def kernel(gt, cls_gt, logits_1, logits_2, selector, it):
    rawA, rawB, tot = _run_ce(cls_gt, logits_1, logits_2, selector)
    rawA = rawA.reshape(4, _N)
    rawB = rawB.reshape(4, _N)
    zero_state = jnp.zeros((8, 16), jnp.int32)
    h0 = _h0_kernel(rawA, rawB)
    h1, st1 = _H1(rawA, rawB, h0, zero_state)
    h2, st2 = _H2(rawA, rawB, h1, st1)
    parts, st3 = _sum_kernel(rawA, rawB, h2, st2)
    v = lax.bitcast_convert_type(st3[:, 0], jnp.float32)
    c_above = st3[:, 1].astype(jnp.float32)
    s_above = jnp.sum(parts.reshape(8, 4 * 16), axis=1)
    tk = s_above + (jnp.float32(_K) - c_above) * v
    per_topk = tk / jnp.float32(_K)
    per_tot = tot[:, 0] / jnp.float32(_N)
    per = jnp.where(it < _START_WARM, per_tot, per_topk)
    return jnp.sum(per) * jnp.float32(0.25)


# speedup vs baseline: 1.8844x; 1.0016x over previous
import functools

import jax
import jax.numpy as jnp
from jax import lax
from jax.experimental import pallas as pl
from jax.experimental.pallas import tpu as pltpu
from jax.experimental.pallas import tpu_sc as plsc

_H = 384
_W = 384
_N = _H * _W
_K = int(_N * 0.15)
_START_WARM = 20000

_MESH = plsc.VectorSubcoreMesh(core_axis_name="c", subcore_axis_name="s")
_CP = pltpu.CompilerParams(needs_layout_passes=False)
_NW = 32
_CHUNK = _N // 4
_NVREG = _CHUNK // 16
_ROUNDS = ((19, 12), (7, 12), (0, 7))


def _raw_ce(L, tgt, full):
    l0, l1, l2 = L[0], L[1], L[2]
    if full:
        m = jnp.maximum(jnp.maximum(l0, l1), l2)
        lse = jnp.log(jnp.exp(l0 - m) + jnp.exp(l1 - m) + jnp.exp(l2 - m)) + m
        lt = jnp.where(tgt == 0, l0, jnp.where(tgt == 1, l1, l2))
    else:
        m = jnp.maximum(l0, l1)
        lse = jnp.log(jnp.exp(l0 - m) + jnp.exp(l1 - m)) + m
        lt = jnp.where(tgt >= 1, l1, l0)
    return jnp.maximum(lse - lt, 0.0)


def _ce_kernel(sel_ref, cls_ref, l1_ref, l2_ref, rawA_ref, rawB_ref, tot_ref):
    j = pl.program_id(0)
    sel_full = sel_ref[j, 1] > 0.5
    for i in (1, 2):
        L = l1_ref[0] if i == 1 else l2_ref[0]
        tgt = cls_ref[0, i]
        raw = lax.cond(sel_full,
                       lambda L=L, tgt=tgt: _raw_ce(L, tgt, True),
                       lambda L=L, tgt=tgt: _raw_ce(L, tgt, False))
        out_ref = rawA_ref if i == 1 else rawB_ref
        out_ref[0] = raw
        tot = jnp.sum(raw).reshape(1, 1)
        tot_ref[pl.ds((i - 1) * 4 + j, 1), :] = jnp.broadcast_to(tot, (1, 128))


def _run_ce(cls_gt, logits_1, logits_2, selector):
    return pl.pallas_call(
        _ce_kernel,
        grid=(4,),
        in_specs=[
            pl.BlockSpec(memory_space=pltpu.SMEM),
            pl.BlockSpec((1, 3, _H, _W), lambda j: (j, 0, 0, 0)),
            pl.BlockSpec((1, 3, _H, _W), lambda j: (j, 0, 0, 0)),
            pl.BlockSpec((1, 3, _H, _W), lambda j: (j, 0, 0, 0)),
        ],
        out_specs=[
            pl.BlockSpec((1, _H, _W), lambda j: (j, 0, 0)),
            pl.BlockSpec((1, _H, _W), lambda j: (j, 0, 0)),
            pl.BlockSpec((8, 128), lambda j: (0, 0)),
        ],
        out_shape=[
            jax.ShapeDtypeStruct((4, _H, _W), jnp.float32),
            jax.ShapeDtypeStruct((4, _H, _W), jnp.float32),
            jax.ShapeDtypeStruct((8, 128), jnp.float32),
        ],
    )(selector, cls_gt, logits_1, logits_2)


def _lane0(vec):
    return jnp.sum(jnp.where(lax.iota(jnp.int32, 16) == 0, vec, 0))


def _lane1(vec):
    return jnp.sum(jnp.where(lax.iota(jnp.int32, 16) == 1, vec, 0))


def _scan_hist(h4_v, kk, nrow):
    iota = lax.iota(jnp.int32, 16)

    def row_c(j):
        sl = pl.ds(j * 16, 16)
        return h4_v[0, sl] + h4_v[1, sl] + h4_v[2, sl] + h4_v[3, sl]

    def scan(t, carry):
        run_c, found, jc, cb = carry
        j = nrow - 1 - t
        rc = jnp.sum(row_c(j))
        take = jnp.logical_and(found == 0, run_c + rc >= kk)
        jc = jnp.where(take, j, jc)
        cb = jnp.where(take, run_c, cb)
        found = jnp.where(take, jnp.int32(1), found)
        return run_c + rc, found, jc, cb

    init = (jnp.int32(0), jnp.int32(0), jnp.int32(0), jnp.int32(0))
    _, _, jc, cb = lax.fori_loop(0, nrow, scan, init)

    rowc = row_c(jc)
    rev_c = lax.rev(rowc, (0,))
    cum_c = plsc.cumsum(rev_c)
    crossed = (cb + cum_c) >= kk
    lstar = plsc.all_reduce_ffs(crossed)
    l0 = jnp.sum(jnp.where(iota == 0, lstar, 0))
    csel = jnp.sum(jnp.where(iota == l0, cum_c, 0))
    crev = jnp.sum(jnp.where(iota == l0, rev_c, 0))
    digit = jc * 16 + 15 - l0
    c_inc = cb + csel - crev
    return digit, c_inc


def _load_chunk(rawA, rawB, data_v, p, q):
    @pl.when(p < 4)
    def _():
        pltpu.sync_copy(rawA.at[p, pl.ds(q * _CHUNK, _CHUNK)], data_v)

    @pl.when(p >= 4)
    def _():
        pltpu.sync_copy(rawB.at[p - 4, pl.ds(q * _CHUNK, _CHUNK)], data_v)


@functools.partial(
    pl.kernel,
    out_type=jax.ShapeDtypeStruct((_NW, 4096), jnp.int32),
    mesh=_MESH, compiler_params=_CP,
    scratch_types=[pltpu.VMEM((_CHUNK,), jnp.float32),
                   pltpu.VMEM((4096,), jnp.int32)])
def _h0_kernel(rawA, rawB, hist_o, data_v, hi_v):
    c = lax.axis_index("c")
    s = lax.axis_index("s")
    wid = c * 16 + s
    _load_chunk(rawA, rawB, data_v, wid // 4, wid % 4)
    zi = jnp.zeros((16,), jnp.int32)

    @plsc.parallel_loop(0, 256, 1, unroll=8)
    def _zero(i):
        hi_v[pl.ds(i * 16, 16)] = zi

    ones_i = jnp.ones((16,), jnp.int32)

    @plsc.parallel_loop(0, _NVREG, 1, unroll=16)
    def _body(i):
        x = data_v[pl.ds(i * 16, 16)]
        u = plsc.bitcast(x, jnp.int32)
        d = lax.shift_right_logical(u, 19)
        plsc.addupdate_scatter(hi_v, [d], ones_i)

    pltpu.sync_copy(hi_v, hist_o.at[wid])


def _make_hist(r):
    shift, width = _ROUNDS[r]
    p_shift, p_width = _ROUNDS[r - 1]
    nb = 1 << width
    nb_prev = 1 << p_width
    first = r == 1

    @functools.partial(
        pl.kernel,
        out_type=[jax.ShapeDtypeStruct((_NW, nb), jnp.int32),
                  jax.ShapeDtypeStruct((8, 16), jnp.int32)],
        mesh=_MESH, compiler_params=_CP,
        scratch_types=[pltpu.VMEM((_CHUNK,), jnp.float32),
                       pltpu.VMEM((4, nb_prev), jnp.int32),
                       pltpu.VMEM((nb,), jnp.int32),
                       pltpu.VMEM((16,), jnp.int32)])
    def hist_kernel(rawA, rawB, prev_hist, prev_state, hist_o, state_o,
                    data_v, h4_v, hi_v, st_v):
        c = lax.axis_index("c")
        s = lax.axis_index("s")
        wid = c * 16 + s
        p = wid // 4
        q = wid % 4
        _load_chunk(rawA, rawB, data_v, p, q)
        pltpu.sync_copy(prev_hist.at[pl.ds(p * 4, 4)], h4_v)

        if first:
            prefix_prev = jnp.int32(0)
            c_prev = jnp.int32(0)
        else:
            pltpu.sync_copy(prev_state.at[p], st_v)
            prefix_prev = _lane0(st_v[...])
            c_prev = _lane1(st_v[...])

        digit, c_inc = _scan_hist(h4_v, jnp.int32(_K) - c_prev, nb_prev // 16)
        prefix = (prefix_prev << p_width) | digit
        c_above = c_prev + c_inc

        zi = jnp.zeros((16,), jnp.int32)

        @plsc.parallel_loop(0, nb // 16, 1, unroll=8)
        def _zero(i):
            hi_v[pl.ds(i * 16, 16)] = zi

        ones_i = jnp.ones((16,), jnp.int32)

        @plsc.parallel_loop(0, _NVREG, 1, unroll=16)
        def _body(i):
            x = data_v[pl.ds(i * 16, 16)]
            u = plsc.bitcast(x, jnp.int32)
            d = lax.shift_right_logical(u, shift) & (nb - 1)
            m = lax.shift_right_logical(u, shift + width) == prefix
            plsc.addupdate_scatter(hi_v, [d], ones_i, mask=m)

        pltpu.sync_copy(hi_v, hist_o.at[wid])

        @pl.when(q == 0)
        def _():
            iota = lax.iota(jnp.int32, 16)
            st_v[...] = jnp.where(iota == 0, prefix,
                                  jnp.where(iota == 1, c_above, 0))
            pltpu.sync_copy(st_v, state_o.at[p])

    return hist_kernel


@functools.partial(
    pl.kernel,
    out_type=[jax.ShapeDtypeStruct((_NW, 16), jnp.float32),
              jax.ShapeDtypeStruct((8, 16), jnp.int32)],
    mesh=_MESH, compiler_params=_CP,
    scratch_types=[pltpu.VMEM((_CHUNK,), jnp.float32),
                   pltpu.VMEM((4, 128), jnp.int32),
                   pltpu.VMEM((16,), jnp.int32),
                   pltpu.VMEM((16,), jnp.float32)])
def _sum_kernel(rawA, rawB, prev_hist, prev_state, parts_o, state_o,
                data_v, h4_v, st_v, tf_v):
    c = lax.axis_index("c")
    s = lax.axis_index("s")
    wid = c * 16 + s
    p = wid // 4
    q = wid % 4
    _load_chunk(rawA, rawB, data_v, p, q)
    pltpu.sync_copy(prev_hist.at[pl.ds(p * 4, 4)], h4_v)
    pltpu.sync_copy(prev_state.at[p], st_v)
    prefix_prev = _lane0(st_v[...])
    c_prev = _lane1(st_v[...])

    digit, c_inc = _scan_hist(h4_v, jnp.int32(_K) - c_prev, 8)
    vbits = (prefix_prev << _ROUNDS[2][1]) | digit
    c_above = c_prev + c_inc

    v = plsc.bitcast(jnp.zeros((16,), jnp.int32) + vbits, jnp.float32)
    zf = jnp.zeros((16,), jnp.float32)

    @plsc.parallel_loop(0, _NVREG // 4, 1, unroll=4, carry=(zf, zf, zf, zf))
    def _accs(i, accs):
        a0, a1, a2, a3 = accs
        base = i * 64
        x0 = data_v[pl.ds(base, 16)]
        x1 = data_v[pl.ds(base + 16, 16)]
        x2 = data_v[pl.ds(base + 32, 16)]
        x3 = data_v[pl.ds(base + 48, 16)]
        return (a0 + jnp.where(x0 > v, x0, zf),
                a1 + jnp.where(x1 > v, x1, zf),
                a2 + jnp.where(x2 > v, x2, zf),
                a3 + jnp.where(x3 > v, x3, zf))
    a0, a1, a2, a3 = _accs
    tf_v[...] = (a0 + a1) + (a2 + a3)
    pltpu.sync_copy(tf_v, parts_o.at[wid])

    @pl.when(q == 0)
    def _():
        iota = lax.iota(jnp.int32, 16)
        st_v[...] = jnp.where(iota == 0, vbits,
                              jnp.where(iota == 1, c_above, 0))
        pltpu.sync_copy(st_v, state_o.at[p])


_H1 = _make_hist(1)
_H2 = _make_hist(2)


def kernel(gt, cls_gt, logits_1, logits_2, selector, it):
    rawA, rawB, tot = _run_ce(cls_gt, logits_1, logits_2, selector)
    rawA = rawA.reshape(4, _N)
    rawB = rawB.reshape(4, _N)
    zero_state = jnp.zeros((8, 16), jnp.int32)
    h0 = _h0_kernel(rawA, rawB)
    h1, st1 = _H1(rawA, rawB, h0, zero_state)
    h2, st2 = _H2(rawA, rawB, h1, st1)
    parts, st3 = _sum_kernel(rawA, rawB, h2, st2)
    v = lax.bitcast_convert_type(st3[:, 0], jnp.float32)
    c_above = st3[:, 1].astype(jnp.float32)
    s_above = jnp.sum(parts.reshape(8, 4 * 16), axis=1)
    tk = s_above + (jnp.float32(_K) - c_above) * v
    per_topk = tk / jnp.float32(_K)
    per_tot = tot[:, 0] / jnp.float32(_N)
    per = jnp.where(it < _START_WARM, per_tot, per_topk)
    return jnp.sum(per) * jnp.float32(0.25)
